# baseline (speedup 1.0000x reference)
import functools

import jax
import jax.numpy as jnp
from jax.experimental import pallas as pl
from jax.experimental.pallas import tpu as pltpu

_EPS = 1e-06
_BLK = 2048


def _ln_body(x_ref, m_ref, w_ref, b_ref, j_ref, o_ref):
    x = x_ref[...]
    c = x.shape[-1]
    j = j_ref[...]
    xb = x.astype(jnp.bfloat16)
    s1 = jax.lax.dot_general(xb, j, (((1,), (0,)), ((), ())),
                             preferred_element_type=jnp.float32)
    s2 = jax.lax.dot_general(xb * xb, j, (((1,), (0,)), ((), ())),
                             preferred_element_type=jnp.float32)
    u = s1 * (1.0 / c)
    v = s2 * (1.0 / c) - u * u
    r = jax.lax.rsqrt(v + _EPS) * m_ref[...]
    o_ref[...] = (x - u) * (r * w_ref[...]) + m_ref[...] * b_ref[...]


def kernel(x, active, weight, bias):
    B, H, W, C = x.shape
    hr = H // active.shape[2]
    wr = W // active.shape[3]
    N = B * H * W
    m = jnp.repeat(jnp.repeat(active[:, 0], hr, axis=1), wr, axis=2)
    m = m.reshape(N, 1).astype(x.dtype)
    x2 = x.reshape(N, C)
    ones_j = jnp.ones((C, C), dtype=jnp.bfloat16)
    grid = N // _BLK
    out = pl.pallas_call(
        _ln_body,
        grid=(grid,),
        in_specs=[
            pl.BlockSpec((_BLK, C), lambda i: (i, 0)),
            pl.BlockSpec((_BLK, 1), lambda i: (i, 0)),
            pl.BlockSpec((1, C), lambda i: (0, 0)),
            pl.BlockSpec((1, C), lambda i: (0, 0)),
            pl.BlockSpec((C, C), lambda i: (0, 0)),
        ],
        out_specs=pl.BlockSpec((_BLK, C), lambda i: (i, 0)),
        out_shape=jax.ShapeDtypeStruct((N, C), x.dtype),
    )(x2, m, weight.reshape(1, C), bias.reshape(1, C), ones_j)
    return out.reshape(B, H, W, C)


# speedup vs baseline: 1.1377x; 1.1377x over previous
import jax
import jax.numpy as jnp
from jax import lax
from jax.experimental import pallas as pl
from jax.experimental.pallas import tpu as pltpu

_EPS = 1e-06
_HB = 16


def _ln_body(bits_ref, x_ref, w_ref, b_ref, j_ref, o_ref):
    bidx = pl.program_id(0)
    hb = pl.program_id(1)
    x = x_ref[...].reshape(_HB * 128, 96)
    xb = x.astype(jnp.bfloat16)
    j = j_ref[...]
    u = lax.dot_general(xb, j, (((1,), (0,)), ((), ())),
                        preferred_element_type=jnp.float32)
    s2 = lax.dot_general(xb * xb, j, (((1,), (0,)), ((), ())),
                         preferred_element_type=jnp.float32)
    v = s2 - u * u
    r = lax.rsqrt(v + _EPS)
    ln = (x - u) * (r * w_ref[...]) + b_ref[...]
    row = lax.broadcasted_iota(jnp.int32, (_HB * 128, 96), 0)
    wc = jnp.right_shift(row, 3) & 15
    s0 = bits_ref[bidx, 2 * hb]
    s1 = bits_ref[bidx, 2 * hb + 1]
    sv = jnp.where(row < 8 * 128, s0, s1)
    bit = jnp.right_shift(sv, wc) & 1
    o_ref[...] = jnp.where(bit != 0, ln, 0.0).reshape(1, _HB, 128, 96)


def kernel(x, active, weight, bias):
    B, H, W, C = x.shape
    bits = jnp.sum(active[:, 0].astype(jnp.int32) << jnp.arange(16, dtype=jnp.int32),
                   axis=-1, dtype=jnp.int32)
    ones_j = jnp.full((C, C), 1.0 / C, dtype=jnp.bfloat16)
    out = pl.pallas_call(
        _ln_body,
        grid=(B, H // _HB),
        in_specs=[
            pl.BlockSpec(memory_space=pltpu.SMEM),
            pl.BlockSpec((1, _HB, 128, C), lambda b, h: (b, h, 0, 0)),
            pl.BlockSpec((1, C), lambda b, h: (0, 0)),
            pl.BlockSpec((1, C), lambda b, h: (0, 0)),
            pl.BlockSpec((C, C), lambda b, h: (0, 0)),
        ],
        out_specs=pl.BlockSpec((1, _HB, 128, C), lambda b, h: (b, h, 0, 0)),
        out_shape=jax.ShapeDtypeStruct((B, H, W, C), x.dtype),
    )(bits, x, weight.reshape(1, C), bias.reshape(1, C), ones_j)
    return out


# speedup vs baseline: 3.4381x; 3.0218x over previous
import jax
import jax.numpy as jnp
from jax import lax
from jax.experimental import pallas as pl
from jax.experimental.pallas import tpu as pltpu

_EPS = 1e-06
_HB = 16


def _ln_body(bits_ref, x_ref, w_ref, b_ref, o_ref):
    bidx = pl.program_id(0)
    hb = pl.program_id(1)
    x = x_ref[...]
    c = x.shape[2]
    u = jnp.mean(x, axis=2, keepdims=True)
    s2 = jnp.mean(x * x, axis=2, keepdims=True)
    v = s2 - u * u
    r = lax.rsqrt(v + _EPS)
    ln = (x - u) * (r * w_ref[...]) + b_ref[...]
    lane = lax.broadcasted_iota(jnp.int32, x.shape, 3)
    h_i = lax.broadcasted_iota(jnp.int32, x.shape, 1)
    s0 = bits_ref[bidx, 2 * hb]
    s1 = bits_ref[bidx, 2 * hb + 1]
    sv = jnp.where(h_i < 8, s0, s1)
    bit = jnp.right_shift(sv, jnp.right_shift(lane, 3)) & 1
    o_ref[...] = jnp.where(bit != 0, ln, 0.0)


def kernel(x, active, weight, bias):
    B, H, W, C = x.shape
    bits = jnp.sum(active[:, 0].astype(jnp.int32) << jnp.arange(16, dtype=jnp.int32),
                   axis=-1, dtype=jnp.int32)
    w_bc = jnp.broadcast_to(weight[:, None], (C, W))
    b_bc = jnp.broadcast_to(bias[:, None], (C, W))
    xt = jnp.transpose(x, (0, 1, 3, 2))
    out_t = pl.pallas_call(
        _ln_body,
        grid=(B, H // _HB),
        in_specs=[
            pl.BlockSpec(memory_space=pltpu.SMEM),
            pl.BlockSpec((1, _HB, C, W), lambda b, h: (b, h, 0, 0)),
            pl.BlockSpec((C, W), lambda b, h: (0, 0)),
            pl.BlockSpec((C, W), lambda b, h: (0, 0)),
        ],
        out_specs=pl.BlockSpec((1, _HB, C, W), lambda b, h: (b, h, 0, 0)),
        out_shape=jax.ShapeDtypeStruct((B, H, C, W), x.dtype),
        compiler_params=pltpu.CompilerParams(
            dimension_semantics=("parallel", "parallel")),
    )(bits, xt, w_bc, b_bc)
    return jnp.transpose(out_t, (0, 1, 3, 2))


# speedup vs baseline: 6.0860x; 1.7702x over previous
import jax
import jax.numpy as jnp
from jax import lax
from jax.experimental import pallas as pl
from jax.experimental.pallas import tpu as pltpu

_EPS = 1e-06
_HB = 64


def _ln_body(bits_ref, x_ref, w_ref, b_ref, o_ref):
    bidx = pl.program_id(0)
    hb = pl.program_id(1)
    x = x_ref[...]
    u = jnp.mean(x, axis=2, keepdims=True)
    s2 = jnp.mean(x * x, axis=2, keepdims=True)
    v = s2 - u * u
    r = lax.rsqrt(v + _EPS)
    ln = (x - u) * (r * w_ref[...]) + b_ref[...]
    wc = jnp.right_shift(lax.broadcasted_iota(jnp.int32, (1, 1, 1, 128), 3), 3)
    for k in range(_HB // 8):
        s = bits_ref[bidx, hb * (_HB // 8) + k]
        bit = jnp.right_shift(s, wc) & 1
        o_ref[0, 8 * k:8 * k + 8] = jnp.where(
            bit[0] != 0, ln[0, 8 * k:8 * k + 8], 0.0)


def kernel(x, active, weight, bias):
    B, H, W, C = x.shape
    bits = jnp.sum(active[:, 0].astype(jnp.int32) << jnp.arange(16, dtype=jnp.int32),
                   axis=-1, dtype=jnp.int32)
    w_bc = jnp.broadcast_to(weight[:, None], (C, W))
    b_bc = jnp.broadcast_to(bias[:, None], (C, W))
    xt = jnp.transpose(x, (0, 1, 3, 2))
    out_t = pl.pallas_call(
        _ln_body,
        grid=(B, H // _HB),
        in_specs=[
            pl.BlockSpec(memory_space=pltpu.SMEM),
            pl.BlockSpec((1, _HB, C, W), lambda b, h: (b, h, 0, 0)),
            pl.BlockSpec((C, W), lambda b, h: (0, 0)),
            pl.BlockSpec((C, W), lambda b, h: (0, 0)),
        ],
        out_specs=pl.BlockSpec((1, _HB, C, W), lambda b, h: (b, h, 0, 0)),
        out_shape=jax.ShapeDtypeStruct((B, H, C, W), x.dtype),
        compiler_params=pltpu.CompilerParams(
            dimension_semantics=("parallel", "parallel")),
    )(bits, xt, w_bc, b_bc)
    return jnp.transpose(out_t, (0, 1, 3, 2))


# speedup vs baseline: 6.3843x; 1.0490x over previous
import jax
import jax.numpy as jnp
from jax import lax
from jax.experimental import pallas as pl
from jax.experimental.pallas import tpu as pltpu

_EPS = 1e-06
_HB = 128


def _ln_body(bits_ref, x_ref, w_ref, b_ref, o_ref):
    bidx = pl.program_id(0)
    hb = pl.program_id(1)
    x = x_ref[...]
    u = jnp.mean(x, axis=2, keepdims=True)
    s2 = jnp.mean(x * x, axis=2, keepdims=True)
    v = s2 - u * u
    r = lax.rsqrt(v + _EPS)
    ln = (x - u) * (r * w_ref[...]) + b_ref[...]
    wc = jnp.right_shift(lax.broadcasted_iota(jnp.int32, (1, 1, 1, 128), 3), 3)
    for k in range(_HB // 8):
        s = bits_ref[bidx, hb * (_HB // 8) + k]
        bit = jnp.right_shift(s, wc) & 1
        o_ref[0, 8 * k:8 * k + 8] = jnp.where(
            bit[0] != 0, ln[0, 8 * k:8 * k + 8], 0.0)


def kernel(x, active, weight, bias):
    B, H, W, C = x.shape
    bits = jnp.sum(active[:, 0].astype(jnp.int32) << jnp.arange(16, dtype=jnp.int32),
                   axis=-1, dtype=jnp.int32)
    w_bc = jnp.broadcast_to(weight[:, None], (C, W))
    b_bc = jnp.broadcast_to(bias[:, None], (C, W))
    xt = jnp.transpose(x, (0, 1, 3, 2))
    out_t = pl.pallas_call(
        _ln_body,
        grid=(B, H // _HB),
        in_specs=[
            pl.BlockSpec(memory_space=pltpu.SMEM),
            pl.BlockSpec((1, _HB, C, W), lambda b, h: (b, h, 0, 0)),
            pl.BlockSpec((C, W), lambda b, h: (0, 0)),
            pl.BlockSpec((C, W), lambda b, h: (0, 0)),
        ],
        out_specs=pl.BlockSpec((1, _HB, C, W), lambda b, h: (b, h, 0, 0)),
        out_shape=jax.ShapeDtypeStruct((B, H, C, W), x.dtype),
        compiler_params=pltpu.CompilerParams(
            dimension_semantics=("parallel", "parallel")),
    )(bits, xt, w_bc, b_bc)
    return jnp.transpose(out_t, (0, 1, 3, 2))
